# baseline (device time: 326054 ns/iter reference)
import jax
import jax.numpy as jnp
from jax import lax
from jax.experimental import pallas as pl
from jax.experimental.pallas import tpu as pltpu

N_X = 2
N_Y = 2
SIZES = [128] * 32
K = len(SIZES)
NCH = 16
USE_DUS = True


def kernel(x):
    m_per, n = x.shape
    half = m_per // 2
    offs = [sum(SIZES[:i]) for i in range(K)]
    och = m_per // NCH

    def body(x_ref, out_ref, vb, in_sems, out_sems,
             send_x, recv_x, send_y, recv_y):
        my_x = lax.axis_index("x")
        my_y = lax.axis_index("y")
        other_x = 1 - my_x
        other_y = 1 - my_y

        barrier_sem = pltpu.get_barrier_semaphore()
        pl.semaphore_signal(
            barrier_sem, inc=1,
            device_id=(other_x, my_y), device_id_type=pl.DeviceIdType.MESH,
        )
        pl.semaphore_signal(
            barrier_sem, inc=1,
            device_id=(my_x, other_y), device_id_type=pl.DeviceIdType.MESH,
        )
        pl.semaphore_wait(barrier_sem, 2)

        xfers = []
        for k in range(K):
            src_row = my_y * half + offs[k]
            dst_row = my_x * m_per + my_y * half + offs[k]
            rdma = pltpu.make_async_remote_copy(
                src_ref=x_ref.at[pl.ds(src_row, SIZES[k]), :],
                dst_ref=out_ref.at[pl.ds(dst_row, SIZES[k]), :],
                send_sem=send_x.at[k],
                recv_sem=recv_x.at[k],
                device_id=(other_x, my_y),
                device_id_type=pl.DeviceIdType.MESH,
            )
            rdma.start()
            xfers.append(rdma)

        ins = []
        outs = []
        if not USE_DUS:
            for k in range(NCH):
                cp = pltpu.make_async_copy(
                    x_ref.at[pl.ds(k * och, och), :], vb.at[k % 2],
                    in_sems.at[k])
                if k < 2:
                    cp.start()
                ins.append(cp)
            for k in range(NCH):
                ins[k].wait()
                cp = pltpu.make_async_copy(
                    vb.at[k % 2],
                    out_ref.at[pl.ds(my_x * m_per + k * och, och), :],
                    out_sems.at[k])
                cp.start()
                outs.append(cp)
                if k + 2 < NCH:
                    outs[k].wait()
                    ins[k + 2].start()

        relays = []
        for k in range(K):
            xfers[k].wait_recv()
            row = other_x * m_per + my_y * half + offs[k]
            relay = pltpu.make_async_remote_copy(
                src_ref=out_ref.at[pl.ds(row, SIZES[k]), :],
                dst_ref=out_ref.at[pl.ds(row, SIZES[k]), :],
                send_sem=send_y.at[k],
                recv_sem=recv_y.at[k],
                device_id=(my_x, other_y),
                device_id_type=pl.DeviceIdType.MESH,
            )
            relay.start()
            relays.append(relay)

        if not USE_DUS:
            for k in range(NCH - 2, NCH):
                outs[k].wait()
        for k in range(K):
            xfers[k].wait_send()
            relays[k].wait_send()
            relays[k].wait_recv()

    out_shape = jax.ShapeDtypeStruct((N_X * m_per, n), x.dtype)
    out = pl.pallas_call(
        body,
        out_shape=out_shape,
        in_specs=[pl.BlockSpec(memory_space=pl.ANY)],
        out_specs=pl.BlockSpec(memory_space=pl.ANY),
        scratch_shapes=[
            pltpu.VMEM((2, m_per // NCH, n), jnp.float32),
            pltpu.SemaphoreType.DMA((NCH,)),
            pltpu.SemaphoreType.DMA((NCH,)),
            pltpu.SemaphoreType.DMA((K,)),
            pltpu.SemaphoreType.DMA((K,)),
            pltpu.SemaphoreType.DMA((K,)),
            pltpu.SemaphoreType.DMA((K,)),
        ],
        compiler_params=pltpu.CompilerParams(collective_id=0),
    )(x)
    if USE_DUS:
        my_x = lax.axis_index("x")
        out = lax.dynamic_update_slice(out, x, (my_x * m_per, 0))
    return out


# device time: 238451 ns/iter; 1.3674x vs baseline; 1.3674x over previous
import jax
import jax.numpy as jnp
from jax import lax
from jax.experimental import pallas as pl
from jax.experimental.pallas import tpu as pltpu

N_X = 2
N_Y = 2
SIZES = [128] * 32
K = len(SIZES)
NCH = 16
USE_DUS = False


def kernel(x):
    m_per, n = x.shape
    half = m_per // 2
    offs = [sum(SIZES[:i]) for i in range(K)]
    och = m_per // NCH

    def body(x_ref, out_ref, vb, in_sems, out_sems,
             send_x, recv_x, send_y, recv_y):
        my_x = lax.axis_index("x")
        my_y = lax.axis_index("y")
        other_x = 1 - my_x
        other_y = 1 - my_y

        barrier_sem = pltpu.get_barrier_semaphore()
        pl.semaphore_signal(
            barrier_sem, inc=1,
            device_id=(other_x, my_y), device_id_type=pl.DeviceIdType.MESH,
        )
        pl.semaphore_signal(
            barrier_sem, inc=1,
            device_id=(my_x, other_y), device_id_type=pl.DeviceIdType.MESH,
        )
        pl.semaphore_wait(barrier_sem, 2)

        xfers = []
        for k in range(K):
            src_row = my_y * half + offs[k]
            dst_row = my_x * m_per + my_y * half + offs[k]
            rdma = pltpu.make_async_remote_copy(
                src_ref=x_ref.at[pl.ds(src_row, SIZES[k]), :],
                dst_ref=out_ref.at[pl.ds(dst_row, SIZES[k]), :],
                send_sem=send_x.at[k],
                recv_sem=recv_x.at[k],
                device_id=(other_x, my_y),
                device_id_type=pl.DeviceIdType.MESH,
            )
            rdma.start()
            xfers.append(rdma)

        ins = []
        outs = []
        if not USE_DUS:
            for k in range(NCH):
                cp = pltpu.make_async_copy(
                    x_ref.at[pl.ds(k * och, och), :], vb.at[k % 2],
                    in_sems.at[k])
                if k < 2:
                    cp.start()
                ins.append(cp)

        def bounce_step(j):
            ins[j].wait()
            cp = pltpu.make_async_copy(
                vb.at[j % 2],
                out_ref.at[pl.ds(my_x * m_per + j * och, och), :],
                out_sems.at[j])
            cp.start()
            outs.append(cp)
            if j + 2 < NCH:
                outs[j].wait()
                ins[j + 2].start()

        relays = []
        for k in range(K):
            xfers[k].wait_recv()
            row = other_x * m_per + my_y * half + offs[k]
            relay = pltpu.make_async_remote_copy(
                src_ref=out_ref.at[pl.ds(row, SIZES[k]), :],
                dst_ref=out_ref.at[pl.ds(row, SIZES[k]), :],
                send_sem=send_y.at[k],
                recv_sem=recv_y.at[k],
                device_id=(my_x, other_y),
                device_id_type=pl.DeviceIdType.MESH,
            )
            relay.start()
            relays.append(relay)
            if not USE_DUS and k % 2 == 0 and k // 2 < NCH:
                bounce_step(k // 2)

        if not USE_DUS:
            for k in range(NCH - 2, NCH):
                outs[k].wait()
        for k in range(K):
            xfers[k].wait_send()
            relays[k].wait_send()
            relays[k].wait_recv()

    out_shape = jax.ShapeDtypeStruct((N_X * m_per, n), x.dtype)
    out = pl.pallas_call(
        body,
        out_shape=out_shape,
        in_specs=[pl.BlockSpec(memory_space=pl.ANY)],
        out_specs=pl.BlockSpec(memory_space=pl.ANY),
        scratch_shapes=[
            pltpu.VMEM((2, m_per // NCH, n), jnp.float32),
            pltpu.SemaphoreType.DMA((NCH,)),
            pltpu.SemaphoreType.DMA((NCH,)),
            pltpu.SemaphoreType.DMA((K,)),
            pltpu.SemaphoreType.DMA((K,)),
            pltpu.SemaphoreType.DMA((K,)),
            pltpu.SemaphoreType.DMA((K,)),
        ],
        compiler_params=pltpu.CompilerParams(collective_id=0),
    )(x)
    if USE_DUS:
        my_x = lax.axis_index("x")
        out = lax.dynamic_update_slice(out, x, (my_x * m_per, 0))
    return out


# device time: 236867 ns/iter; 1.3765x vs baseline; 1.0067x over previous
import jax
import jax.numpy as jnp
from jax import lax
from jax.experimental import pallas as pl
from jax.experimental.pallas import tpu as pltpu

N_X = 2
N_Y = 2
SIZES = [64] * 64
K = len(SIZES)
NCH = 16
USE_DUS = False


def kernel(x):
    m_per, n = x.shape
    half = m_per // 2
    offs = [sum(SIZES[:i]) for i in range(K)]
    och = m_per // NCH

    def body(x_ref, out_ref, vb, in_sems, out_sems,
             send_x, recv_x, send_y, recv_y):
        my_x = lax.axis_index("x")
        my_y = lax.axis_index("y")
        other_x = 1 - my_x
        other_y = 1 - my_y

        barrier_sem = pltpu.get_barrier_semaphore()
        pl.semaphore_signal(
            barrier_sem, inc=1,
            device_id=(other_x, my_y), device_id_type=pl.DeviceIdType.MESH,
        )
        pl.semaphore_signal(
            barrier_sem, inc=1,
            device_id=(my_x, other_y), device_id_type=pl.DeviceIdType.MESH,
        )
        pl.semaphore_wait(barrier_sem, 2)

        xfers = []
        for k in range(K):
            src_row = my_y * half + offs[k]
            dst_row = my_x * m_per + my_y * half + offs[k]
            rdma = pltpu.make_async_remote_copy(
                src_ref=x_ref.at[pl.ds(src_row, SIZES[k]), :],
                dst_ref=out_ref.at[pl.ds(dst_row, SIZES[k]), :],
                send_sem=send_x.at[k],
                recv_sem=recv_x.at[k],
                device_id=(other_x, my_y),
                device_id_type=pl.DeviceIdType.MESH,
            )
            rdma.start()
            xfers.append(rdma)

        ins = []
        outs = []
        if not USE_DUS:
            for k in range(NCH):
                cp = pltpu.make_async_copy(
                    x_ref.at[pl.ds(k * och, och), :], vb.at[k % 2],
                    in_sems.at[k])
                if k < 2:
                    cp.start()
                ins.append(cp)

        def bounce_step(j):
            ins[j].wait()
            cp = pltpu.make_async_copy(
                vb.at[j % 2],
                out_ref.at[pl.ds(my_x * m_per + j * och, och), :],
                out_sems.at[j])
            cp.start()
            outs.append(cp)
            if j + 2 < NCH:
                outs[j].wait()
                ins[j + 2].start()

        relays = []
        for k in range(K):
            xfers[k].wait_recv()
            row = other_x * m_per + my_y * half + offs[k]
            relay = pltpu.make_async_remote_copy(
                src_ref=out_ref.at[pl.ds(row, SIZES[k]), :],
                dst_ref=out_ref.at[pl.ds(row, SIZES[k]), :],
                send_sem=send_y.at[k],
                recv_sem=recv_y.at[k],
                device_id=(my_x, other_y),
                device_id_type=pl.DeviceIdType.MESH,
            )
            relay.start()
            relays.append(relay)
            step = K // NCH
            if not USE_DUS and k % step == 0 and k // step < NCH:
                bounce_step(k // step)

        if not USE_DUS:
            for k in range(NCH - 2, NCH):
                outs[k].wait()
        for k in range(K):
            xfers[k].wait_send()
            relays[k].wait_send()
            relays[k].wait_recv()

    out_shape = jax.ShapeDtypeStruct((N_X * m_per, n), x.dtype)
    out = pl.pallas_call(
        body,
        out_shape=out_shape,
        in_specs=[pl.BlockSpec(memory_space=pl.ANY)],
        out_specs=pl.BlockSpec(memory_space=pl.ANY),
        scratch_shapes=[
            pltpu.VMEM((2, m_per // NCH, n), jnp.float32),
            pltpu.SemaphoreType.DMA((NCH,)),
            pltpu.SemaphoreType.DMA((NCH,)),
            pltpu.SemaphoreType.DMA((K,)),
            pltpu.SemaphoreType.DMA((K,)),
            pltpu.SemaphoreType.DMA((K,)),
            pltpu.SemaphoreType.DMA((K,)),
        ],
        compiler_params=pltpu.CompilerParams(collective_id=0),
    )(x)
    if USE_DUS:
        my_x = lax.axis_index("x")
        out = lax.dynamic_update_slice(out, x, (my_x * m_per, 0))
    return out
